# baseline (speedup 1.0000x reference)
import jax
import jax.numpy as jnp
from jax import lax
from jax.experimental import pallas as pl
from jax.experimental.pallas import tpu as pltpu
from jax.experimental.pallas import tpu_sc as plsc

NUM_EMBEDDINGS = 1000000
D = 64
B = 16384
S = 50

NC = 2
NS = 16
NW = NC * NS
B_PER_W = B // NW
CHUNK_B = 8
N_CHUNKS = B_PER_W // CHUNK_B
IDX_PER_CHUNK = CHUNK_B * S
GATHER_W = 80
N_GATHERS = IDX_PER_CHUNK // GATHER_W
L = 16


def _body(tok_hbm, table_hbm, out_hbm, idx2d, idx_all, rows_v, out_all, gsem, osem):
    wid = lax.axis_index("s") * NC + lax.axis_index("c")

    lanes = jax.lax.iota(jnp.int32, L)
    HALF = B_PER_W // 2

    for h in range(2):
        pltpu.sync_copy(
            tok_hbm.at[:, pl.ds(wid * B_PER_W + h * HALF, HALF)], idx2d
        )

        @pl.loop(0, HALF)
        def _transpose(b):
            b_vec = jnp.broadcast_to(b, (L,))
            for k in range((S + L - 1) // L):
                s_vec = lanes + (k * L)
                mask = s_vec < S
                ids = plsc.load_gather(idx2d, [s_vec, b_vec], mask=mask)
                plsc.store_scatter(
                    idx_all, [(h * HALF + b) * S + s_vec], ids, mask=mask
                )

    def fire(cc, p):
        for j in range(N_GATHERS):
            pltpu.async_copy(
                table_hbm.at[
                    idx_all.at[pl.ds(cc * IDX_PER_CHUNK + j * GATHER_W, GATHER_W)]
                ],
                rows_v.at[p, pl.ds(j * GATHER_W, GATHER_W)],
                gsem.at[p],
            )

    def drain(p):
        pltpu.make_async_copy(
            table_hbm.at[pl.ds(0, IDX_PER_CHUNK)], rows_v.at[p], gsem.at[p]
        ).wait()

    fire(0, 0)

    @pl.loop(0, N_CHUNKS, step=2)
    def _chunks(c):
        for par in range(2):
            cc = c + par

            @pl.when(cc + 1 < N_CHUNKS)
            def _():
                fire(cc + 1, 1 - par)

            drain(par)

            @pl.loop(0, CHUNK_B)
            def _row(b):
                base = b * S
                accs = [rows_v[par, base, pl.ds(d * L, L)] for d in range(D // L)]
                for s in range(1, S):
                    for d in range(D // L):
                        accs[d] = accs[d] + rows_v[par, base + s, pl.ds(d * L, L)]
                orow = cc * CHUNK_B + b
                for d in range(D // L):
                    out_all[orow, pl.ds(d * L, L)] = accs[d]

    pltpu.async_copy(out_all, out_hbm.at[pl.ds(wid * B_PER_W, B_PER_W)], osem).wait()


@jax.jit
def kernel(token_ids, table):
    tok_t = token_ids.astype(jnp.int32).T
    mesh = plsc.VectorSubcoreMesh(core_axis_name="c", subcore_axis_name="s")
    k = pl.kernel(
        _body,
        out_type=jax.ShapeDtypeStruct((B, D), jnp.float32),
        mesh=mesh,
        scratch_types=[
            pltpu.VMEM((S, B_PER_W // 2), jnp.int32),
            pltpu.VMEM((B_PER_W * S,), jnp.int32),
            pltpu.VMEM((2, IDX_PER_CHUNK, D), jnp.float32),
            pltpu.VMEM((B_PER_W, D), jnp.float32),
            pltpu.SemaphoreType.DMA((2,)),
            pltpu.SemaphoreType.DMA,
        ],
        compiler_params=pltpu.CompilerParams(
            use_tc_tiling_on_sc=False, needs_layout_passes=False
        ),
    )
    return k(tok_t, table)
